# baseline (speedup 1.0000x reference)
import functools

import jax
import jax.numpy as jnp
from jax.experimental import pallas as pl
from jax.experimental.pallas import tpu as pltpu

TILE_N = 2048


def _mm_kernel(x_ref, em_ref, o_ref):
    o_ref[...] = jax.lax.dot_general(
        x_ref[...], em_ref[...],
        dimension_numbers=(((1,), (1,)), ((), ())),
        preferred_element_type=jnp.float32,
    )


@functools.partial(jax.jit, static_argnames=())
def kernel(inputs, targets, em):
    del targets
    m, k = inputs.shape
    n = em.shape[0]
    grid = (pl.cdiv(n, TILE_N),)
    out = pl.pallas_call(
        _mm_kernel,
        grid=grid,
        in_specs=[
            pl.BlockSpec((m, k), lambda i: (0, 0)),
            pl.BlockSpec((TILE_N, k), lambda i: (i, 0)),
        ],
        out_specs=pl.BlockSpec((m, TILE_N), lambda i: (0, i)),
        out_shape=jax.ShapeDtypeStruct((m, n), jnp.float32),
        compiler_params=pltpu.CompilerParams(
            dimension_semantics=("arbitrary",),
        ),
    )(inputs, em)
    return out


# speedup vs baseline: 1.0038x; 1.0038x over previous
import functools

import jax
import jax.numpy as jnp
from jax.experimental import pallas as pl
from jax.experimental.pallas import tpu as pltpu

TILE_N = 4096


def _mm_kernel(x_ref, em_ref, o_ref):
    o_ref[...] = jax.lax.dot_general(
        x_ref[...], em_ref[...],
        dimension_numbers=(((1,), (1,)), ((), ())),
        preferred_element_type=jnp.float32,
    )


@functools.partial(jax.jit, static_argnames=())
def kernel(inputs, targets, em):
    del targets
    m, k = inputs.shape
    n = em.shape[0]
    grid = (pl.cdiv(n, TILE_N),)
    out = pl.pallas_call(
        _mm_kernel,
        grid=grid,
        in_specs=[
            pl.BlockSpec((m, k), lambda i: (0, 0)),
            pl.BlockSpec((TILE_N, k), lambda i: (i, 0)),
        ],
        out_specs=pl.BlockSpec((m, TILE_N), lambda i: (0, i)),
        out_shape=jax.ShapeDtypeStruct((m, n), jnp.float32),
        compiler_params=pltpu.CompilerParams(
            dimension_semantics=("arbitrary",),
        ),
    )(inputs, em)
    return out
